# baseline (speedup 1.0000x reference)
import jax
import jax.numpy as jnp
from jax.experimental import pallas as pl
from jax.experimental.pallas import tpu as pltpu
from jax.experimental.pallas import tpu_sc as plsc

M = 4096
N = 4096
BI = 256
_NB = M // BI
_NROW = 2 * _NB
_SUB = N // 128
_NIDX = _NROW * _SUB
_WIN = 128


def _halo_gather(d, idx):
    @pl.kernel(
        out_type=jax.ShapeDtypeStruct((_NIDX, 128), jnp.float32),
        mesh=plsc.VectorSubcoreMesh(core_axis_name="c", subcore_axis_name="s"),
    )
    def gather_kernel(d_hbm, i_hbm, o_hbm):
        def body(i_vmem, o_vmem):
            pltpu.sync_copy(d_hbm.at[i_vmem.at[0]], o_vmem)

        pltpu.emit_pipeline(
            body,
            grid=(_NIDX // _WIN,),
            in_specs=[pl.BlockSpec((1, _WIN), index_map=lambda i: (0, i))],
            out_specs=[pl.BlockSpec((_WIN, 128), index_map=lambda i: (i, 0))],
            core_axis_name=("c", "s"),
            dimension_semantics=(pltpu.PARALLEL,),
        )(i_hbm, o_hbm)

    return gather_kernel(d.reshape(M * _SUB, 128), idx)


def _stencil_kernel(thr_ref, topA_ref, topD_ref, bot_ref, d_ref, out_ref):
    i = pl.program_id(0)
    t = thr_ref[0]
    c = d_ref[...]
    topA = topA_ref[0]
    topD = topD_ref[0]
    bot = bot_ref[0]

    inf = jnp.float32(jnp.inf)
    infcol = jnp.full((BI, 1), inf, jnp.float32)
    infcol1 = jnp.full((1, 1), inf, jnp.float32)

    cL = jnp.concatenate([infcol, c[:, :-1]], axis=1)
    tAL = jnp.concatenate([infcol1, topA[:, :-1]], axis=1)
    bL = jnp.concatenate([infcol1, bot[:, :-1]], axis=1)
    cR = jnp.concatenate([c[:, 1:], c[:, :1]], axis=1)
    tDR = jnp.concatenate([topD[:, 1:], topD[:, :1]], axis=1)
    cRC = jnp.concatenate([c[:, 1:], infcol], axis=1)
    bRC = jnp.concatenate([bot[:, 1:], infcol1], axis=1)

    upAL = jnp.concatenate([tAL, cL[:-1, :]], axis=0)
    dnL = jnp.concatenate([cL[1:, :], bL], axis=0)
    dnRC = jnp.concatenate([cRC[1:, :], bRC], axis=0)
    upDR = jnp.concatenate([tDR, cR[:-1, :]], axis=0)

    cA = jnp.abs(upAL - c) <= t
    cB = jnp.abs(dnL - c) <= t
    cC = jnp.abs(dnRC - c) <= t
    cD = jnp.abs(upDR - c) <= t

    row = jax.lax.broadcasted_iota(jnp.int32, (BI, N), 0)
    lanes = jax.lax.broadcasted_iota(jnp.int32, (BI, N), 1)
    mB = (row >= 1 - i * BI) & (lanes <= N - 2)

    combined = (cA | (cB & mB)) | (cC | cD)
    out_ref[...] = combined.astype(jnp.int8).T != 0


def kernel(d_noised, threshold):
    nb = M // BI
    starts = jnp.arange(nb) * BI
    inf_row = jnp.full((1, N), jnp.inf, jnp.float32)
    rowidx = jnp.concatenate(
        [(starts - 1) % M, starts[:-1] + BI, jnp.zeros((1,), starts.dtype)]
    ).astype(jnp.int32)
    idx = (rowidx[:, None] * _SUB + jnp.arange(_SUB, dtype=jnp.int32)).reshape(
        1, _NIDX
    )
    g = _halo_gather(d_noised, idx).reshape(_NROW, N)
    topD_rows = g[0:nb]
    topA_rows = jnp.concatenate([inf_row, g[1:nb]], axis=0)
    bot_rows = jnp.concatenate([g[nb : 2 * nb - 1], inf_row], axis=0)
    thr = jnp.reshape(threshold, (1,))

    out = pl.pallas_call(
        _stencil_kernel,
        grid=(nb,),
        in_specs=[
            pl.BlockSpec(memory_space=pltpu.SMEM),
            pl.BlockSpec((1, 1, N), lambda i: (i, 0, 0)),
            pl.BlockSpec((1, 1, N), lambda i: (i, 0, 0)),
            pl.BlockSpec((1, 1, N), lambda i: (i, 0, 0)),
            pl.BlockSpec((BI, N), lambda i: (i, 0)),
        ],
        out_specs=pl.BlockSpec((N, BI), lambda i: (0, i)),
        out_shape=jax.ShapeDtypeStruct((N, M), jnp.bool_),
        compiler_params=pltpu.CompilerParams(
            dimension_semantics=("arbitrary",),
        ),
    )(
        thr,
        topA_rows.reshape(nb, 1, N),
        topD_rows.reshape(nb, 1, N),
        bot_rows.reshape(nb, 1, N),
        d_noised,
    )
    return out


# speedup vs baseline: 1.3514x; 1.3514x over previous
import jax
import jax.numpy as jnp
from jax.experimental import pallas as pl
from jax.experimental.pallas import tpu as pltpu
from jax.experimental.pallas import tpu_sc as plsc

M = 4096
N = 4096
BI = 256
_NB = M // BI
_NROW = 2 * _NB
_SUB = N // 128
_NIDX = _NROW * _SUB
_WIN = 128


def _halo_gather(d):
    @pl.kernel(
        out_type=jax.ShapeDtypeStruct((_NROW, N), jnp.float32),
        mesh=plsc.ScalarSubcoreMesh(axis_name="c", num_cores=2),
        scratch_types=[pltpu.SemaphoreType.DMA],
    )
    def gather_kernel(d_hbm, o_hbm, sem):
        core = jax.lax.axis_index("c")
        handles = []
        for k in range(_NB):
            r = core * _NB + k
            src = jnp.where(
                r < _NB, (r * BI + M - 1) % M, ((r - (_NB - 1)) * BI) % M
            )
            handles.append(pltpu.async_copy(d_hbm.at[src], o_hbm.at[r], sem))
        for h in handles:
            h.wait()

    return gather_kernel(d)


def _stencil_kernel(thr_ref, topA_ref, topD_ref, bot_ref, d_ref, out_ref):
    i = pl.program_id(0)
    t = thr_ref[0]
    c = d_ref[...]
    topA = topA_ref[0]
    topD = topD_ref[0]
    bot = bot_ref[0]

    inf = jnp.float32(jnp.inf)
    infcol = jnp.full((BI, 1), inf, jnp.float32)
    infcol1 = jnp.full((1, 1), inf, jnp.float32)

    cL = jnp.concatenate([infcol, c[:, :-1]], axis=1)
    tAL = jnp.concatenate([infcol1, topA[:, :-1]], axis=1)
    bL = jnp.concatenate([infcol1, bot[:, :-1]], axis=1)
    cR = jnp.concatenate([c[:, 1:], c[:, :1]], axis=1)
    tDR = jnp.concatenate([topD[:, 1:], topD[:, :1]], axis=1)
    cRC = jnp.concatenate([c[:, 1:], infcol], axis=1)
    bRC = jnp.concatenate([bot[:, 1:], infcol1], axis=1)

    upAL = jnp.concatenate([tAL, cL[:-1, :]], axis=0)
    dnL = jnp.concatenate([cL[1:, :], bL], axis=0)
    dnRC = jnp.concatenate([cRC[1:, :], bRC], axis=0)
    upDR = jnp.concatenate([tDR, cR[:-1, :]], axis=0)

    cA = jnp.abs(upAL - c) <= t
    cB = jnp.abs(dnL - c) <= t
    cC = jnp.abs(dnRC - c) <= t
    cD = jnp.abs(upDR - c) <= t

    row = jax.lax.broadcasted_iota(jnp.int32, (BI, N), 0)
    lanes = jax.lax.broadcasted_iota(jnp.int32, (BI, N), 1)
    mB = (row >= 1 - i * BI) & (lanes <= N - 2)

    combined = (cA | (cB & mB)) | (cC | cD)
    out_ref[...] = combined.astype(jnp.int8).T != 0


def kernel(d_noised, threshold):
    nb = M // BI
    starts = jnp.arange(nb) * BI
    inf_row = jnp.full((1, N), jnp.inf, jnp.float32)
    del starts
    g = _halo_gather(d_noised)
    topD_rows = g[0:nb]
    topA_rows = jnp.concatenate([inf_row, g[1:nb]], axis=0)
    bot_rows = jnp.concatenate([g[nb : 2 * nb - 1], inf_row], axis=0)
    thr = jnp.reshape(threshold, (1,))

    out = pl.pallas_call(
        _stencil_kernel,
        grid=(nb,),
        in_specs=[
            pl.BlockSpec(memory_space=pltpu.SMEM),
            pl.BlockSpec((1, 1, N), lambda i: (i, 0, 0)),
            pl.BlockSpec((1, 1, N), lambda i: (i, 0, 0)),
            pl.BlockSpec((1, 1, N), lambda i: (i, 0, 0)),
            pl.BlockSpec((BI, N), lambda i: (i, 0)),
        ],
        out_specs=pl.BlockSpec((N, BI), lambda i: (0, i)),
        out_shape=jax.ShapeDtypeStruct((N, M), jnp.bool_),
        compiler_params=pltpu.CompilerParams(
            dimension_semantics=("arbitrary",),
        ),
    )(
        thr,
        topA_rows.reshape(nb, 1, N),
        topD_rows.reshape(nb, 1, N),
        bot_rows.reshape(nb, 1, N),
        d_noised,
    )
    return out


# speedup vs baseline: 1.6461x; 1.2181x over previous
import jax
import jax.numpy as jnp
from jax.experimental import pallas as pl
from jax.experimental.pallas import tpu as pltpu
M = 4096
N = 4096
BI = 256


def _stencil_kernel(thr_ref, topA_ref, topD_ref, bot_ref, d_ref, out_ref):
    i = pl.program_id(0)
    t = thr_ref[0]
    c = d_ref[...]
    topA = topA_ref[0]
    topD = topD_ref[0]
    bot = bot_ref[0]

    inf = jnp.float32(jnp.inf)
    infcol = jnp.full((BI, 1), inf, jnp.float32)
    infcol1 = jnp.full((1, 1), inf, jnp.float32)

    cL = jnp.concatenate([infcol, c[:, :-1]], axis=1)
    tAL = jnp.concatenate([infcol1, topA[:, :-1]], axis=1)
    bL = jnp.concatenate([infcol1, bot[:, :-1]], axis=1)
    cR = jnp.concatenate([c[:, 1:], c[:, :1]], axis=1)
    tDR = jnp.concatenate([topD[:, 1:], topD[:, :1]], axis=1)
    cRC = jnp.concatenate([c[:, 1:], infcol], axis=1)
    bRC = jnp.concatenate([bot[:, 1:], infcol1], axis=1)

    upAL = jnp.concatenate([tAL, cL[:-1, :]], axis=0)
    dnL = jnp.concatenate([cL[1:, :], bL], axis=0)
    dnRC = jnp.concatenate([cRC[1:, :], bRC], axis=0)
    upDR = jnp.concatenate([tDR, cR[:-1, :]], axis=0)

    cA = jnp.abs(upAL - c) <= t
    cB = jnp.abs(dnL - c) <= t
    cC = jnp.abs(dnRC - c) <= t
    cD = jnp.abs(upDR - c) <= t

    row = jax.lax.broadcasted_iota(jnp.int32, (BI, N), 0)
    lanes = jax.lax.broadcasted_iota(jnp.int32, (BI, N), 1)
    mB = (row >= 1 - i * BI) & (lanes <= N - 2)

    combined = (cA | (cB & mB)) | (cC | cD)
    out_ref[...] = combined.astype(jnp.int8).T != 0


def kernel(d_noised, threshold):
    nb = M // BI
    starts = jnp.arange(nb) * BI
    inf_row = jnp.full((1, N), jnp.inf, jnp.float32)
    topD_rows = jnp.take(d_noised, (starts - 1) % M, axis=0)
    topA_rows = jnp.concatenate([inf_row, topD_rows[1:]], axis=0)
    bot_rows = jnp.concatenate(
        [jnp.take(d_noised, starts[:-1] + BI, axis=0), inf_row], axis=0
    )
    thr = jnp.reshape(threshold, (1,))

    out = pl.pallas_call(
        _stencil_kernel,
        grid=(nb,),
        in_specs=[
            pl.BlockSpec(memory_space=pltpu.SMEM),
            pl.BlockSpec((1, 1, N), lambda i: (i, 0, 0)),
            pl.BlockSpec((1, 1, N), lambda i: (i, 0, 0)),
            pl.BlockSpec((1, 1, N), lambda i: (i, 0, 0)),
            pl.BlockSpec((BI, N), lambda i: (i, 0)),
        ],
        out_specs=pl.BlockSpec((N, BI), lambda i: (0, i)),
        out_shape=jax.ShapeDtypeStruct((N, M), jnp.bool_),
        compiler_params=pltpu.CompilerParams(
            dimension_semantics=("arbitrary",),
        ),
    )(
        thr,
        topA_rows.reshape(nb, 1, N),
        topD_rows.reshape(nb, 1, N),
        bot_rows.reshape(nb, 1, N),
        d_noised,
    )
    return out


# speedup vs baseline: 1.7662x; 1.0730x over previous
import jax
import jax.numpy as jnp
from jax.experimental import pallas as pl
from jax.experimental.pallas import tpu as pltpu
M = 4096
N = 4096
BI = 256


def _stencil_kernel(thr_ref, topA_ref, topD_ref, bot_ref, d_ref, out_ref):
    i = pl.program_id(0)
    t = thr_ref[0]
    c = d_ref[...]
    topA = topA_ref[0]
    topD = topD_ref[0]
    bot = bot_ref[0]

    inf = jnp.float32(jnp.inf)
    infcol = jnp.full((BI, 1), inf, jnp.float32)
    infcol1 = jnp.full((1, 1), inf, jnp.float32)

    cL = jnp.concatenate([infcol, c[:, :-1]], axis=1)
    tAL = jnp.concatenate([infcol1, topA[:, :-1]], axis=1)
    bL = jnp.concatenate([infcol1, bot[:, :-1]], axis=1)
    cR = jnp.concatenate([c[:, 1:], c[:, :1]], axis=1)
    tDR = jnp.concatenate([topD[:, 1:], topD[:, :1]], axis=1)
    cRC = jnp.concatenate([c[:, 1:], infcol], axis=1)
    bRC = jnp.concatenate([bot[:, 1:], infcol1], axis=1)

    upAL = jnp.concatenate([tAL, cL[:-1, :]], axis=0)
    dnL = jnp.concatenate([cL[1:, :], bL], axis=0)
    dnRC = jnp.concatenate([cRC[1:, :], bRC], axis=0)
    upDR = jnp.concatenate([tDR, cR[:-1, :]], axis=0)

    cA = jnp.abs(upAL - c) <= t
    cB = jnp.abs(dnL - c) <= t
    cC = jnp.abs(dnRC - c) <= t
    cD = jnp.abs(upDR - c) <= t

    lanes = jax.lax.broadcasted_iota(jnp.int32, (BI, N), 1)
    lm = lanes <= N - 2

    def _finish(mB):
        combined = (cA | (cB & mB)) | (cC | cD)
        out_ref[...] = combined.astype(jnp.int8).T != 0

    @pl.when(i == 0)
    def _():
        row = jax.lax.broadcasted_iota(jnp.int32, (BI, N), 0)
        _finish((row >= 1) & lm)

    @pl.when(i > 0)
    def _():
        _finish(lm)


def kernel(d_noised, threshold):
    nb = M // BI
    starts = jnp.arange(nb) * BI
    inf_row = jnp.full((1, N), jnp.inf, jnp.float32)
    topD_rows = jnp.take(d_noised, (starts - 1) % M, axis=0)
    topA_rows = jnp.concatenate([inf_row, topD_rows[1:]], axis=0)
    bot_rows = jnp.concatenate(
        [jnp.take(d_noised, starts[:-1] + BI, axis=0), inf_row], axis=0
    )
    thr = jnp.reshape(threshold, (1,))

    out = pl.pallas_call(
        _stencil_kernel,
        grid=(nb,),
        in_specs=[
            pl.BlockSpec(memory_space=pltpu.SMEM),
            pl.BlockSpec((1, 1, N), lambda i: (i, 0, 0)),
            pl.BlockSpec((1, 1, N), lambda i: (i, 0, 0)),
            pl.BlockSpec((1, 1, N), lambda i: (i, 0, 0)),
            pl.BlockSpec((BI, N), lambda i: (i, 0)),
        ],
        out_specs=pl.BlockSpec((N, BI), lambda i: (0, i)),
        out_shape=jax.ShapeDtypeStruct((N, M), jnp.bool_),
        compiler_params=pltpu.CompilerParams(
            dimension_semantics=("arbitrary",),
        ),
    )(
        thr,
        topA_rows.reshape(nb, 1, N),
        topD_rows.reshape(nb, 1, N),
        bot_rows.reshape(nb, 1, N),
        d_noised,
    )
    return out


# speedup vs baseline: 1.7687x; 1.0014x over previous
import jax
import jax.numpy as jnp
from jax.experimental import pallas as pl
from jax.experimental.pallas import tpu as pltpu
M = 4096
N = 4096
BI = 256


def _stencil_kernel(thr_ref, topA_ref, topD_ref, bot_ref, d_ref, out_ref):
    i = pl.program_id(0)
    t = thr_ref[0]
    c = d_ref[...]
    topA = topA_ref[0]
    topD = topD_ref[0]
    bot = bot_ref[0]

    inf = jnp.float32(jnp.inf)
    infcol = jnp.full((BI, 1), inf, jnp.float32)
    infcol1 = jnp.full((1, 1), inf, jnp.float32)

    cL = jnp.concatenate([infcol, c[:, :-1]], axis=1)
    tAL = jnp.concatenate([infcol1, topA[:, :-1]], axis=1)
    cLB = jnp.concatenate([infcol, c[:, : N - 2], infcol], axis=1)
    bLB = jnp.concatenate([infcol1, bot[:, : N - 2], infcol1], axis=1)
    cR = jnp.concatenate([c[:, 1:], c[:, :1]], axis=1)
    tDR = jnp.concatenate([topD[:, 1:], topD[:, :1]], axis=1)
    cRC = jnp.concatenate([c[:, 1:], infcol], axis=1)
    bRC = jnp.concatenate([bot[:, 1:], infcol1], axis=1)

    upAL = jnp.concatenate([tAL, cL[:-1, :]], axis=0)
    dnL = jnp.concatenate([cLB[1:, :], bLB], axis=0)
    dnRC = jnp.concatenate([cRC[1:, :], bRC], axis=0)
    upDR = jnp.concatenate([tDR, cR[:-1, :]], axis=0)

    cA = jnp.abs(upAL - c) <= t
    cB = jnp.abs(dnL - c) <= t
    cC = jnp.abs(dnRC - c) <= t
    cD = jnp.abs(upDR - c) <= t

    @pl.when(i == 0)
    def _():
        row = jax.lax.broadcasted_iota(jnp.int32, (BI, N), 0)
        combined = (cA | (cB & (row >= 1))) | (cC | cD)
        out_ref[...] = combined.astype(jnp.int8).T != 0

    @pl.when(i > 0)
    def _():
        combined = (cA | cB) | (cC | cD)
        out_ref[...] = combined.astype(jnp.int8).T != 0


def kernel(d_noised, threshold):
    nb = M // BI
    starts = jnp.arange(nb) * BI
    inf_row = jnp.full((1, N), jnp.inf, jnp.float32)
    topD_rows = jnp.take(d_noised, (starts - 1) % M, axis=0)
    topA_rows = jnp.concatenate([inf_row, topD_rows[1:]], axis=0)
    bot_rows = jnp.concatenate(
        [jnp.take(d_noised, starts[:-1] + BI, axis=0), inf_row], axis=0
    )
    thr = jnp.reshape(threshold, (1,))

    out = pl.pallas_call(
        _stencil_kernel,
        grid=(nb,),
        in_specs=[
            pl.BlockSpec(memory_space=pltpu.SMEM),
            pl.BlockSpec((1, 1, N), lambda i: (i, 0, 0)),
            pl.BlockSpec((1, 1, N), lambda i: (i, 0, 0)),
            pl.BlockSpec((1, 1, N), lambda i: (i, 0, 0)),
            pl.BlockSpec((BI, N), lambda i: (i, 0)),
        ],
        out_specs=pl.BlockSpec((N, BI), lambda i: (0, i)),
        out_shape=jax.ShapeDtypeStruct((N, M), jnp.bool_),
        compiler_params=pltpu.CompilerParams(
            dimension_semantics=("arbitrary",),
        ),
    )(
        thr,
        topA_rows.reshape(nb, 1, N),
        topD_rows.reshape(nb, 1, N),
        bot_rows.reshape(nb, 1, N),
        d_noised,
    )
    return out
